# baseline (speedup 1.0000x reference)
import functools

import jax
import jax.numpy as jnp
from jax import lax
from jax.experimental import pallas as pl
from jax.experimental.pallas import tpu as pltpu
from jax.experimental.pallas import tpu_sc as plsc

N_CORES = 2
N_SUBCORES = 16
N_WORKERS = N_CORES * N_SUBCORES
CHUNK = 128
LANES = 16


def _make_bpr_kernel(B, D):
    rows_per_w = B // N_WORKERS
    n_chunks = rows_per_w // CHUNK
    n_groups = rows_per_w // LANES
    mesh = plsc.VectorSubcoreMesh(core_axis_name="c", subcore_axis_name="s")

    @functools.partial(
        pl.kernel,
        mesh=mesh,
        compiler_params=pltpu.CompilerParams(
            needs_layout_passes=False, use_tc_tiling_on_sc=False),
        out_type=(
            jax.ShapeDtypeStruct((B,), jnp.float32),
            jax.ShapeDtypeStruct((B,), jnp.float32),
        ),
        scratch_types=[
            pltpu.VMEM((n_chunks, CHUNK), jnp.int32),
            pltpu.VMEM((n_chunks, CHUNK), jnp.int32),
            pltpu.VMEM((n_chunks, CHUNK), jnp.int32),
            pltpu.VMEM((rows_per_w, D), jnp.float32),
            pltpu.VMEM((rows_per_w, D), jnp.float32),
            pltpu.VMEM((rows_per_w, D), jnp.float32),
            pltpu.VMEM((rows_per_w,), jnp.float32),
            pltpu.VMEM((rows_per_w,), jnp.float32),
            pltpu.SemaphoreType.DMA,
        ],
    )
    def kern(user_hbm, item_hbm, u_hbm, i_hbm, j_hbm,
             pos_hbm, neg_hbm,
             u_v, i_v, j_v, ue_v, ie_v, je_v, pos_v, neg_v, sem):
        wid = lax.axis_index("s") * N_CORES + lax.axis_index("c")
        base = wid * rows_per_w

        pltpu.sync_copy(u_hbm.at[wid], u_v)
        pltpu.sync_copy(i_hbm.at[wid], i_v)
        pltpu.sync_copy(j_hbm.at[wid], j_v)

        copies = []
        for c in range(n_chunks):
            sl = pl.ds(c * CHUNK, CHUNK)
            copies.append(pltpu.async_copy(user_hbm.at[u_v.at[c]], ue_v.at[sl], sem))
            copies.append(pltpu.async_copy(item_hbm.at[i_v.at[c]], ie_v.at[sl], sem))
            copies.append(pltpu.async_copy(item_hbm.at[j_v.at[c]], je_v.at[sl], sem))
        for cp in copies:
            cp.wait()

        lanes = lax.iota(jnp.int32, LANES)

        def group_body(g, carry):
            rows = g * LANES + lanes
            acc_p = jnp.zeros((LANES,), jnp.float32)
            acc_n = jnp.zeros((LANES,), jnp.float32)
            for d in range(D):
                col = jnp.full((LANES,), d, jnp.int32)
                ue = plsc.load_gather(ue_v, [rows, col])
                ie = plsc.load_gather(ie_v, [rows, col])
                je = plsc.load_gather(je_v, [rows, col])
                acc_p = acc_p + ue * ie
                acc_n = acc_n + ue * je
            pos_v[pl.ds(g * LANES, LANES)] = acc_p
            neg_v[pl.ds(g * LANES, LANES)] = acc_n
            return carry

        lax.fori_loop(0, n_groups, group_body, 0)

        pltpu.sync_copy(pos_v, pos_hbm.at[pl.ds(base, rows_per_w)])
        pltpu.sync_copy(neg_v, neg_hbm.at[pl.ds(base, rows_per_w)])

    return kern


def kernel(u, i, j, labels, user_embed, item_embed):
    B = u.shape[0]
    D = user_embed.shape[1]
    u32 = u.astype(jnp.int32).reshape(N_WORKERS, -1, CHUNK)
    i32 = i.astype(jnp.int32).reshape(N_WORKERS, -1, CHUNK)
    j32 = j.astype(jnp.int32).reshape(N_WORKERS, -1, CHUNK)
    pos, neg = _make_bpr_kernel(B, D)(user_embed, item_embed, u32, i32, j32)
    return pos.reshape(B, 1), neg.reshape(B, 1)
